# baseline (speedup 1.0000x reference)
import jax
import jax.numpy as jnp
from jax import lax
from jax.experimental import pallas as pl
from jax.experimental.pallas import tpu as pltpu
from jax.experimental.pallas import tpu_sc as plsc

N_POINTS = 1_000_000
GRID = 1024
D = 32
K = 4
B = 65536

NC = 2
NS = 16
L = 16
NW = NC * NS
PER_W = B // NW
CHUNK = 128
NCH = PER_W // CHUNK
GPC = CHUNK // L


def _rsqrt(x):
    i = lax.bitcast_convert_type(x, jnp.int32)
    i = jnp.int32(0x5F3759DF) - lax.shift_right_arithmetic(i, 1)
    y = lax.bitcast_convert_type(i, jnp.float32)
    for _ in range(3):
        y = y * (1.5 - 0.5 * x * y * y)
    return y


def _body(pos_hbm, pts_hbm, emb_hbm, nm_hbm, out_hbm,
          pos_v, qx, qy, cellidx, nidx, pxi, pyi, ptx, pty, emb_rows,
          wn, out_v, sem_nm, sem_pt, sem_emb):
    wid = lax.axis_index("s") * NC + lax.axis_index("c")
    base = wid * PER_W

    iota = lax.iota(jnp.int32, L)

    pltpu.sync_copy(pos_hbm.at[pl.ds(base * 2, PER_W * 2)], pos_v)

    def p1(i, _):
        rows2 = (i * L + iota) * 2
        px = plsc.load_gather(pos_v, [rows2])
        py = plsc.load_gather(pos_v, [rows2 + 1])
        ix = px.astype(jnp.int32)
        iy = py.astype(jnp.int32)
        cell4 = (ix * GRID + iy) * K
        c = i // GPC
        o = (i % GPC) * L
        for k in range(K):
            cellidx[k, c, pl.ds(o, L)] = cell4 + k
        qx[pl.ds(i * L, L)] = ix.astype(jnp.float32)
        qy[pl.ds(i * L, L)] = iy.astype(jnp.float32)
        return 0

    lax.fori_loop(0, PER_W // L, p1, 0)

    def chunk(c, _):
        nm_cp = [pltpu.async_copy(nm_hbm.at[cellidx.at[k, c]], nidx.at[k],
                                  sem_nm) for k in range(K)]
        for cp in nm_cp:
            cp.wait()

        for k in range(K):
            for o in range(GPC):
                nb = nidx[k, pl.ds(o * L, L)]
                nb2 = nb + nb
                pxi[k, pl.ds(o * L, L)] = nb2
                pyi[k, pl.ds(o * L, L)] = nb2 + 1

        cps = []
        for k in range(K):
            cps.append(pltpu.async_copy(
                pts_hbm.at[pxi.at[k]], ptx.at[k], sem_pt))
            cps.append(pltpu.async_copy(
                pts_hbm.at[pyi.at[k]], pty.at[k], sem_pt))
            cps.append(pltpu.async_copy(
                emb_hbm.at[nidx.at[k]],
                emb_rows.at[pl.ds(k * CHUNK, CHUNK)], sem_emb))
        for cp in cps:
            cp.wait()

        def wstep(o, _):
            qs = pl.ds(c * CHUNK + o * L, L)
            qxv = qx[qs]
            qyv = qy[qs]
            ws = []
            for k in range(K):
                s = pl.ds(o * L, L)
                dx = ptx[k, s] - qxv
                dy = pty[k, s] - qyv
                d2 = dx * dx + dy * dy
                dist = d2 * _rsqrt(d2)
                ws.append(1.0 / (dist + 1e-6))
            inv = 1.0 / (ws[0] + ws[1] + ws[2] + ws[3])
            for k in range(K):
                wn[k, pl.ds(o * L, L)] = ws[k] * inv
            return 0

        lax.fori_loop(0, GPC, wstep, 0)

        def comb(g, _):
            s = pl.ds(g * L, L)
            wv = [wn[k, s] for k in range(K)]
            for lane in range(L):
                q = g * L + lane
                w = [wv[k][lane] for k in range(K)]
                e = [emb_rows.at[k * CHUNK + q] for k in range(K)]
                for h in range(2):
                    hs = pl.ds(h * L, L)
                    acc = (w[0] * e[0][hs] + w[1] * e[1][hs]
                           + w[2] * e[2][hs] + w[3] * e[3][hs])
                    out_v[q, hs] = acc
            return 0

        lax.fori_loop(0, GPC, comb, 0)

        pltpu.sync_copy(out_v, out_hbm.at[pl.ds(base + c * CHUNK, CHUNK)])
        return 0

    lax.fori_loop(0, NCH, chunk, 0)


@jax.jit
def _run(pos_flat, pts_flat, embeddings, nm_flat):
    mesh = plsc.VectorSubcoreMesh(core_axis_name="c", subcore_axis_name="s")
    f = pl.kernel(
        _body,
        out_type=jax.ShapeDtypeStruct((B, D), jnp.float32),
        mesh=mesh,
        compiler_params=pltpu.CompilerParams(
            needs_layout_passes=False, use_tc_tiling_on_sc=False),
        scratch_types=[
            pltpu.VMEM((PER_W * 2,), jnp.float32),
            pltpu.VMEM((PER_W,), jnp.float32),
            pltpu.VMEM((PER_W,), jnp.float32),
            pltpu.VMEM((K, NCH, CHUNK), jnp.int32),
            pltpu.VMEM((K, CHUNK), jnp.int32),
            pltpu.VMEM((K, CHUNK), jnp.int32),
            pltpu.VMEM((K, CHUNK), jnp.int32),
            pltpu.VMEM((K, CHUNK), jnp.float32),
            pltpu.VMEM((K, CHUNK), jnp.float32),
            pltpu.VMEM((K * CHUNK, D), jnp.float32),
            pltpu.VMEM((K, CHUNK), jnp.float32),
            pltpu.VMEM((CHUNK, D), jnp.float32),
            pltpu.SemaphoreType.DMA,
            pltpu.SemaphoreType.DMA,
            pltpu.SemaphoreType.DMA,
        ],
    )
    return f(pos_flat, pts_flat, embeddings, nm_flat)


def kernel(position, positions, embeddings, neighbor_map):
    return _run(position.reshape(B * 2), positions.reshape(N_POINTS * 2),
                embeddings, neighbor_map.reshape(GRID * GRID * K))


# speedup vs baseline: 1.0093x; 1.0093x over previous
import jax
import jax.numpy as jnp
from jax import lax
from jax.experimental import pallas as pl
from jax.experimental.pallas import tpu as pltpu
from jax.experimental.pallas import tpu_sc as plsc

N_POINTS = 1_000_000
GRID = 1024
D = 32
K = 4
B = 65536

NC = 2
NS = 16
L = 16
NW = NC * NS
PER_W = B // NW
CHUNK = 128
NCH = PER_W // CHUNK
GPC = CHUNK // L


def _rsqrt(x):
    i = lax.bitcast_convert_type(x, jnp.int32)
    i = jnp.int32(0x5F3759DF) - lax.shift_right_arithmetic(i, 1)
    y = lax.bitcast_convert_type(i, jnp.float32)
    for _ in range(3):
        y = y * (1.5 - 0.5 * x * y * y)
    return y


def _body(pos_hbm, pts_hbm, emb_hbm, nm_hbm, out_hbm,
          pos_v, qx, qy, cellidx, nidx, pxi, pyi, ptx, pty, emb_rows,
          wn, out_v, sem_nm, sem_pt0, sem_pt1, sem_emb0, sem_emb1):
    wid = lax.axis_index("s") * NC + lax.axis_index("c")
    base = wid * PER_W

    iota = lax.iota(jnp.int32, L)

    pltpu.sync_copy(pos_hbm.at[pl.ds(base * 2, PER_W * 2)], pos_v)

    def p1(i, _):
        rows2 = (i * L + iota) * 2
        px = plsc.load_gather(pos_v, [rows2])
        py = plsc.load_gather(pos_v, [rows2 + 1])
        ix = px.astype(jnp.int32)
        iy = py.astype(jnp.int32)
        cell4 = (ix * GRID + iy) * K
        c = i // GPC
        o = (i % GPC) * L
        for k in range(K):
            cellidx[k, c, pl.ds(o, L)] = cell4 + k
        qx[pl.ds(i * L, L)] = ix.astype(jnp.float32)
        qy[pl.ds(i * L, L)] = iy.astype(jnp.float32)
        return 0

    lax.fori_loop(0, GPC, p1, 0)
    for k in range(K):
        pltpu.async_copy(nm_hbm.at[cellidx.at[k, 0]], nidx.at[0, k], sem_nm)
    lax.fori_loop(GPC, PER_W // L, p1, 0)

    def _streams(p, sem_pt, sem_emb, launch):
        for k in range(K):
            for src, dst, sem in (
                    (pts_hbm.at[pxi.at[p, k]], ptx.at[p, k], sem_pt),
                    (pts_hbm.at[pyi.at[p, k]], pty.at[p, k], sem_pt),
                    (emb_hbm.at[nidx.at[p, k]],
                     emb_rows.at[p, pl.ds(k * CHUNK, CHUNK)], sem_emb)):
                if launch:
                    pltpu.async_copy(src, dst, sem)
                else:
                    pltpu.make_async_copy(src, dst, sem).wait()

    def fire_big(par):
        @pl.when(par == 0)
        def _():
            _streams(0, sem_pt0, sem_emb0, True)

        @pl.when(par == 1)
        def _():
            _streams(1, sem_pt1, sem_emb1, True)

    def wait_big(par):
        @pl.when(par == 0)
        def _():
            _streams(0, sem_pt0, sem_emb0, False)

        @pl.when(par == 1)
        def _():
            _streams(1, sem_pt1, sem_emb1, False)

    def compute(c, par):
        wait_big(par)

        def wstep(o, _):
            qs = pl.ds(c * CHUNK + o * L, L)
            qxv = qx[qs]
            qyv = qy[qs]
            ws = []
            for k in range(K):
                s = pl.ds(o * L, L)
                dx = ptx[par, k, s] - qxv
                dy = pty[par, k, s] - qyv
                d2 = dx * dx + dy * dy
                dist = d2 * _rsqrt(d2)
                ws.append(1.0 / (dist + 1e-6))
            inv = 1.0 / (ws[0] + ws[1] + ws[2] + ws[3])
            for k in range(K):
                wn[k, pl.ds(o * L, L)] = ws[k] * inv
            return 0

        lax.fori_loop(0, GPC, wstep, 0)

        def comb(g, _):
            s = pl.ds(g * L, L)
            wv = [wn[k, s] for k in range(K)]
            for lane in range(L):
                q = g * L + lane
                w = [wv[k][lane] for k in range(K)]
                e = [emb_rows.at[par, k * CHUNK + q] for k in range(K)]
                for h in range(2):
                    hs = pl.ds(h * L, L)
                    acc = (w[0] * e[0][hs] + w[1] * e[1][hs]
                           + w[2] * e[2][hs] + w[3] * e[3][hs])
                    out_v[q, hs] = acc
            return 0

        lax.fori_loop(0, GPC, comb, 0)
        pltpu.sync_copy(out_v, out_hbm.at[pl.ds(base + c * CHUNK, CHUNK)])

    def chunk(c, _):
        par = c & 1

        for k in range(K):
            pltpu.make_async_copy(nm_hbm.at[cellidx.at[k, c]],
                                  nidx.at[par, k], sem_nm).wait()
        for k in range(K):
            for o in range(GPC):
                s = pl.ds(o * L, L)
                nb = nidx[par, k, s]
                nb2 = nb + nb
                pxi[par, k, s] = nb2
                pyi[par, k, s] = nb2 + 1

        fire_big(par)

        @pl.when(c < NCH - 1)
        def _():
            for k in range(K):
                pltpu.async_copy(nm_hbm.at[cellidx.at[k, c + 1]],
                                 nidx.at[1 - par, k], sem_nm)

        @pl.when(c > 0)
        def _():
            compute(c - 1, 1 - par)

        return 0

    lax.fori_loop(0, NCH, chunk, 0)
    compute(NCH - 1, (NCH - 1) & 1)


@jax.jit
def _run(pos_flat, pts_flat, embeddings, nm_flat):
    mesh = plsc.VectorSubcoreMesh(core_axis_name="c", subcore_axis_name="s")
    f = pl.kernel(
        _body,
        out_type=jax.ShapeDtypeStruct((B, D), jnp.float32),
        mesh=mesh,
        compiler_params=pltpu.CompilerParams(
            needs_layout_passes=False, use_tc_tiling_on_sc=False),
        scratch_types=[
            pltpu.VMEM((PER_W * 2,), jnp.float32),
            pltpu.VMEM((PER_W,), jnp.float32),
            pltpu.VMEM((PER_W,), jnp.float32),
            pltpu.VMEM((K, NCH, CHUNK), jnp.int32),
            pltpu.VMEM((2, K, CHUNK), jnp.int32),
            pltpu.VMEM((2, K, CHUNK), jnp.int32),
            pltpu.VMEM((2, K, CHUNK), jnp.int32),
            pltpu.VMEM((2, K, CHUNK), jnp.float32),
            pltpu.VMEM((2, K, CHUNK), jnp.float32),
            pltpu.VMEM((2, K * CHUNK, D), jnp.float32),
            pltpu.VMEM((K, CHUNK), jnp.float32),
            pltpu.VMEM((CHUNK, D), jnp.float32),
            pltpu.SemaphoreType.DMA,
            pltpu.SemaphoreType.DMA,
            pltpu.SemaphoreType.DMA,
            pltpu.SemaphoreType.DMA,
            pltpu.SemaphoreType.DMA,
        ],
    )
    return f(pos_flat, pts_flat, embeddings, nm_flat)


def kernel(position, positions, embeddings, neighbor_map):
    return _run(position.reshape(B * 2), positions.reshape(N_POINTS * 2),
                embeddings, neighbor_map.reshape(GRID * GRID * K))
